# baseline (speedup 1.0000x reference)
import functools

import jax
import jax.numpy as jnp
from jax import lax
from jax.experimental import pallas as pl
from jax.experimental.pallas import tpu as pltpu
from jax.experimental.pallas import tpu_sc as plsc

_B, _S, _D = 16, 2048, 1024
_L = 16
_NW = 32
_CH = 16
_GB = _CH * _D

_B_SC = 2
_RPW = _B_SC * _S // _NW
_NG = _RPW // _CH
_WPB = _NW // _B_SC


def _zero_row(buf, j):
    def _z(i, _):
        buf[j, pl.ds(i * 4 * _L, _L)] = jnp.zeros((_L,), jnp.float32)
        buf[j, pl.ds(i * 4 * _L + _L, _L)] = jnp.zeros((_L,), jnp.float32)
        buf[j, pl.ds(i * 4 * _L + 2 * _L, _L)] = jnp.zeros((_L,), jnp.float32)
        buf[j, pl.ds(i * 4 * _L + 3 * _L, _L)] = jnp.zeros((_L,), jnp.float32)
        return 0
    lax.fori_loop(0, _D // (4 * _L), _z, 0)


_NCHAN = 4


def _fire_gather(g, x3, buf, sem_g, first, out3, sem_s, b, s0):
    @pl.when(jnp.logical_not(first))
    def _():
        pltpu.make_async_copy(buf, out3.at[0, pl.ds(0, _CH)], sem_s).wait()

    pltpu.async_copy(x3.at[b, pl.ds(s0 + g * _CH, _CH)], buf, sem_g)


def _finish_group(g, out3, buf, sem_g, sem_s, mask_v, s0, b, length):
    pltpu.make_async_copy(out3.at[0, pl.ds(0, _CH)], buf, sem_g).wait()

    m = mask_v[pl.ds(s0 + g * _CH, _L)]
    for j in range(_CH):
        keep_j = (m[j] != 0) & (s0 + g * _CH + j < length)

        @pl.when(jnp.logical_not(keep_j))
        def _(j=j):
            _zero_row(buf, j)

    pltpu.async_copy(buf, out3.at[b, pl.ds(s0 + g * _CH, _CH)], sem_s)


def _sc_body(x3, mask_i, out3, mask_v, buf0, buf1, buf2, buf3, zeros_v,
             sem_g0, sem_g1, sem_g2, sem_g3,
             sem_s0, sem_s1, sem_s2, sem_s3, sem_z):
    cid = lax.axis_index("c")
    sid = lax.axis_index("s")
    wid = sid * 2 + cid
    b = wid // _WPB
    s0 = (wid % _WPB) * _RPW
    base = wid * _RPW
    lanes = lax.iota(jnp.int32, _L)

    for j in range(_CH):
        _zero_row(zeros_v, j)

    pltpu.sync_copy(mask_i.at[b], mask_v.at[pl.ds(0, _S)])

    def _len_step(i, acc):
        return acc + mask_v[pl.ds(i * _L, _L)]

    acc = lax.fori_loop(0, _S // _L, _len_step, jnp.zeros((_L,), jnp.int32))
    mask_v[pl.ds(_S + _L, _L)] = jnp.zeros((_L,), jnp.int32)
    for k in (8, 4, 2, 1):
        mask_v[pl.ds(_S, _L)] = acc
        acc = acc + mask_v[pl.ds(_S + k, _L)]
    length = acc[0]

    galive = jnp.clip((length - s0 + _CH - 1) // _CH, 0, _NG)

    del lanes
    bufs = (buf0, buf1, buf2, buf3)
    gsems = (sem_g0, sem_g1, sem_g2, sem_g3)
    ssems = (sem_s0, sem_s1, sem_s2, sem_s3)

    def _quad(t, _):
        for c in range(_NCHAN):
            g = _NCHAN * t + c

            @pl.when(g < galive)
            def _(g=g, c=c):
                _fire_gather(g, x3, bufs[c], gsems[c], t == 0, out3,
                             ssems[c], b, s0)

        for c in range(_NCHAN):
            g = _NCHAN * t + c

            @pl.when(g < galive)
            def _(g=g, c=c):
                _finish_group(g, out3, bufs[c], gsems[c], ssems[c],
                              mask_v, s0, b, length)

        return 0
    lax.fori_loop(0, _NG // _NCHAN, _quad, 0)

    def _ztail(g, _):
        pltpu.async_copy(zeros_v, out3.at[b, pl.ds(s0 + g * _CH, _CH)], sem_z)
        return 0
    lax.fori_loop(galive, _NG, _ztail, 0)

    for c in range(_NCHAN):
        @pl.when(galive > c)
        def _(c=c):
            pltpu.make_async_copy(bufs[c], out3.at[0, pl.ds(0, _CH)],
                                  ssems[c]).wait()

    def _zdrain(g, _):
        pltpu.make_async_copy(zeros_v, out3.at[0, pl.ds(0, _CH)], sem_z).wait()
        return 0
    lax.fori_loop(galive, _NG, _zdrain, 0)


_sc_call = functools.partial(
    pl.kernel,
    out_type=jax.ShapeDtypeStruct((_B_SC, _S, _D), jnp.float32),
    mesh=plsc.VectorSubcoreMesh(core_axis_name="c", subcore_axis_name="s"),
    scratch_types=[
        pltpu.VMEM((_S + 2 * _L,), jnp.int32),
        pltpu.VMEM((_CH, _D), jnp.float32),
        pltpu.VMEM((_CH, _D), jnp.float32),
        pltpu.VMEM((_CH, _D), jnp.float32),
        pltpu.VMEM((_CH, _D), jnp.float32),
        pltpu.VMEM((_CH, _D), jnp.float32),
        pltpu.SemaphoreType.DMA,
        pltpu.SemaphoreType.DMA,
        pltpu.SemaphoreType.DMA,
        pltpu.SemaphoreType.DMA,
        pltpu.SemaphoreType.DMA,
        pltpu.SemaphoreType.DMA,
        pltpu.SemaphoreType.DMA,
        pltpu.SemaphoreType.DMA,
        pltpu.SemaphoreType.DMA,
    ],
)(_sc_body)


_S_BLK = 2048


def _tc_body(mask_ref, x_ref, o_ref):
    j = pl.program_id(1)
    m_row = mask_ref[0, 0, :]
    length = jnp.sum(m_row)
    m_blk = mask_ref[0, 0, pl.ds(j * _S_BLK, _S_BLK)]
    pos = jax.lax.broadcasted_iota(jnp.int32, (_S_BLK, 1), 0) + j * _S_BLK
    keep = (m_blk.reshape(_S_BLK, 1) != 0) & (pos < length)
    o_ref[0] = x_ref[0] * keep.astype(jnp.float32)


def kernel(x, mask):
    mi = mask.astype(jnp.int32)
    out_sc = _sc_call(x, mi)
    m3 = mi.reshape(_B, 1, _S)
    out_tc = pl.pallas_call(
        _tc_body,
        grid=(_B - _B_SC, _S // _S_BLK),
        in_specs=[
            pl.BlockSpec((1, 1, _S), lambda b, j: (b + _B_SC, 0, 0)),
            pl.BlockSpec((1, _S_BLK, _D), lambda b, j: (b + _B_SC, j, 0)),
        ],
        out_specs=pl.BlockSpec((1, _S_BLK, _D), lambda b, j: (b, j, 0)),
        out_shape=jax.ShapeDtypeStruct((_B - _B_SC, _S, _D), jnp.float32),
    )(m3, x)
    return jnp.concatenate([out_sc, out_tc], axis=0)


# speedup vs baseline: 1.0087x; 1.0087x over previous
import functools

import jax
import jax.numpy as jnp
from jax import lax
from jax.experimental import pallas as pl
from jax.experimental.pallas import tpu as pltpu
from jax.experimental.pallas import tpu_sc as plsc

_B, _S, _D = 16, 2048, 1024
_L = 16
_NW = 32
_CH = 16
_GB = _CH * _D

_B_SC = 4
_RPW = _B_SC * _S // _NW
_NG = _RPW // _CH
_WPB = _NW // _B_SC


def _zero_row(buf, j):
    def _z(i, _):
        buf[j, pl.ds(i * 4 * _L, _L)] = jnp.zeros((_L,), jnp.float32)
        buf[j, pl.ds(i * 4 * _L + _L, _L)] = jnp.zeros((_L,), jnp.float32)
        buf[j, pl.ds(i * 4 * _L + 2 * _L, _L)] = jnp.zeros((_L,), jnp.float32)
        buf[j, pl.ds(i * 4 * _L + 3 * _L, _L)] = jnp.zeros((_L,), jnp.float32)
        return 0
    lax.fori_loop(0, _D // (4 * _L), _z, 0)


_NCHAN = 4


def _fire_gather(g, x3, buf, sem_g, first, out3, sem_s, b, s0):
    @pl.when(jnp.logical_not(first))
    def _():
        pltpu.make_async_copy(buf, out3.at[0, pl.ds(0, _CH)], sem_s).wait()

    pltpu.async_copy(x3.at[b, pl.ds(s0 + g * _CH, _CH)], buf, sem_g)


def _finish_group(g, out3, buf, sem_g, sem_s, mask_v, s0, b, length):
    pltpu.make_async_copy(out3.at[0, pl.ds(0, _CH)], buf, sem_g).wait()

    m = mask_v[pl.ds(s0 + g * _CH, _L)]
    for j in range(_CH):
        keep_j = (m[j] != 0) & (s0 + g * _CH + j < length)

        @pl.when(jnp.logical_not(keep_j))
        def _(j=j):
            _zero_row(buf, j)

    pltpu.async_copy(buf, out3.at[b, pl.ds(s0 + g * _CH, _CH)], sem_s)


def _sc_body(x3, mask_i, out3, mask_v, buf0, buf1, buf2, buf3, zeros_v,
             sem_g0, sem_g1, sem_g2, sem_g3,
             sem_s0, sem_s1, sem_s2, sem_s3, sem_z):
    cid = lax.axis_index("c")
    sid = lax.axis_index("s")
    wid = sid * 2 + cid
    b = wid // _WPB
    s0 = (wid % _WPB) * _RPW
    base = wid * _RPW
    lanes = lax.iota(jnp.int32, _L)

    for j in range(_CH):
        _zero_row(zeros_v, j)

    pltpu.sync_copy(mask_i.at[b], mask_v.at[pl.ds(0, _S)])

    def _len_step(i, acc):
        return acc + mask_v[pl.ds(i * _L, _L)]

    acc = lax.fori_loop(0, _S // _L, _len_step, jnp.zeros((_L,), jnp.int32))
    mask_v[pl.ds(_S + _L, _L)] = jnp.zeros((_L,), jnp.int32)
    for k in (8, 4, 2, 1):
        mask_v[pl.ds(_S, _L)] = acc
        acc = acc + mask_v[pl.ds(_S + k, _L)]
    length = acc[0]

    galive = jnp.clip((length - s0 + _CH - 1) // _CH, 0, _NG)

    del lanes
    bufs = (buf0, buf1, buf2, buf3)
    gsems = (sem_g0, sem_g1, sem_g2, sem_g3)
    ssems = (sem_s0, sem_s1, sem_s2, sem_s3)

    def _quad(t, _):
        for c in range(_NCHAN):
            g = _NCHAN * t + c

            @pl.when(g < galive)
            def _(g=g, c=c):
                _fire_gather(g, x3, bufs[c], gsems[c], t == 0, out3,
                             ssems[c], b, s0)

        for c in range(_NCHAN):
            g = _NCHAN * t + c

            @pl.when(g < galive)
            def _(g=g, c=c):
                _finish_group(g, out3, bufs[c], gsems[c], ssems[c],
                              mask_v, s0, b, length)

        return 0
    lax.fori_loop(0, _NG // _NCHAN, _quad, 0)

    def _ztail(g, _):
        pltpu.async_copy(zeros_v, out3.at[b, pl.ds(s0 + g * _CH, _CH)], sem_z)
        return 0
    lax.fori_loop(galive, _NG, _ztail, 0)

    for c in range(_NCHAN):
        @pl.when(galive > c)
        def _(c=c):
            pltpu.make_async_copy(bufs[c], out3.at[0, pl.ds(0, _CH)],
                                  ssems[c]).wait()

    def _zdrain(g, _):
        pltpu.make_async_copy(zeros_v, out3.at[0, pl.ds(0, _CH)], sem_z).wait()
        return 0
    lax.fori_loop(galive, _NG, _zdrain, 0)


_sc_call = functools.partial(
    pl.kernel,
    out_type=jax.ShapeDtypeStruct((_B_SC, _S, _D), jnp.float32),
    mesh=plsc.VectorSubcoreMesh(core_axis_name="c", subcore_axis_name="s"),
    scratch_types=[
        pltpu.VMEM((_S + 2 * _L,), jnp.int32),
        pltpu.VMEM((_CH, _D), jnp.float32),
        pltpu.VMEM((_CH, _D), jnp.float32),
        pltpu.VMEM((_CH, _D), jnp.float32),
        pltpu.VMEM((_CH, _D), jnp.float32),
        pltpu.VMEM((_CH, _D), jnp.float32),
        pltpu.SemaphoreType.DMA,
        pltpu.SemaphoreType.DMA,
        pltpu.SemaphoreType.DMA,
        pltpu.SemaphoreType.DMA,
        pltpu.SemaphoreType.DMA,
        pltpu.SemaphoreType.DMA,
        pltpu.SemaphoreType.DMA,
        pltpu.SemaphoreType.DMA,
        pltpu.SemaphoreType.DMA,
    ],
)(_sc_body)


_S_BLK = 2048


def _tc_body(mask_ref, x_ref, o_ref):
    j = pl.program_id(1)
    m_row = mask_ref[0, 0, :]
    length = jnp.sum(m_row)
    m_blk = mask_ref[0, 0, pl.ds(j * _S_BLK, _S_BLK)]
    pos = jax.lax.broadcasted_iota(jnp.int32, (_S_BLK, 1), 0) + j * _S_BLK
    keep = (m_blk.reshape(_S_BLK, 1) != 0) & (pos < length)
    o_ref[0] = x_ref[0] * keep.astype(jnp.float32)


def kernel(x, mask):
    mi = mask.astype(jnp.int32)
    out_sc = _sc_call(x, mi)
    m3 = mi.reshape(_B, 1, _S)
    out_tc = pl.pallas_call(
        _tc_body,
        grid=(_B - _B_SC, _S // _S_BLK),
        in_specs=[
            pl.BlockSpec((1, 1, _S), lambda b, j: (b + _B_SC, 0, 0)),
            pl.BlockSpec((1, _S_BLK, _D), lambda b, j: (b + _B_SC, j, 0)),
        ],
        out_specs=pl.BlockSpec((1, _S_BLK, _D), lambda b, j: (b, j, 0)),
        out_shape=jax.ShapeDtypeStruct((_B - _B_SC, _S, _D), jnp.float32),
    )(m3, x)
    return jnp.concatenate([out_sc, out_tc], axis=0)
